# Initial kernel scaffold; baseline (speedup 1.0000x reference)
import jax
import jax.numpy as jnp
from jax.experimental import pallas as pl


def kernel(x, table):
    raise NotImplementedError("write your pallas kernel here")


# speedup vs baseline: 10.2140x; 10.2140x over previous
import functools

import jax
import jax.numpy as jnp
from jax import lax
from jax.experimental import pallas as pl
from jax.experimental.pallas import tpu as pltpu
from jax.experimental.pallas import tpu_sc as plsc

BITS = 17
MAXV = 2 ** (BITS - 1) - 1
SIGN = 1 << (BITS - 1)
LANES = 16
NW = 32


def kernel(x, table):
    del table
    B, S = x.shape
    N = B * S
    xf = x.reshape(N).astype(jnp.int32)

    n_w = N // NW
    C = 2048
    n_it = n_w // C

    mesh = plsc.VectorSubcoreMesh(core_axis_name="c", subcore_axis_name="s")

    @functools.partial(
        pl.kernel,
        mesh=mesh,
        out_type=jax.ShapeDtypeStruct((N * BITS,), jnp.float32),
        scratch_types=[
            pltpu.VMEM((C,), jnp.int32),
            pltpu.VMEM((C * BITS,), jnp.float32),
        ],
        compiler_params=pltpu.CompilerParams(needs_layout_passes=False),
    )
    def run(x_hbm, out_hbm, xb, ob):
        wid = lax.axis_index("s") * 2 + lax.axis_index("c")
        base = wid * n_w

        lane = lax.iota(jnp.int32, LANES)
        div_tabs = [(lane + t * LANES) // BITS for t in range(BITS)]
        shift_tabs = [
            (BITS - 1) - ((lane + t * LANES) % BITS) for t in range(BITS)
        ]

        def chunk_body(it, carry):
            off = base + it * C
            pltpu.sync_copy(x_hbm.at[pl.ds(off, C)], xb)

            def pre(g, carry2):
                sl = pl.ds(g * LANES, LANES)
                xv = xb[sl]
                xc = jnp.clip(xv, -MAXV, MAXV)
                xb[sl] = jnp.abs(xc) + jnp.where(xc < 0, SIGN, 0)
                return carry2

            lax.fori_loop(0, C // LANES, pre, 0)

            def grp(g, carry2):
                gbase = g * LANES
                obase = g * (LANES * BITS)
                for t in range(BITS):
                    u = plsc.load_gather(xb, [gbase + div_tabs[t]])
                    bit = (u >> shift_tabs[t]) & 1
                    ob[pl.ds(obase + t * LANES, LANES)] = bit.astype(jnp.float32)
                return carry2

            lax.fori_loop(0, C // LANES, grp, 0)
            pltpu.sync_copy(ob, out_hbm.at[pl.ds(off * BITS, C * BITS)])
            return carry

        lax.fori_loop(0, n_it, chunk_body, 0)

    out = run(xf)
    return out.reshape(B, S, BITS)


# speedup vs baseline: 11.7423x; 1.1496x over previous
import functools

import jax
import jax.numpy as jnp
from jax import lax
from jax.experimental import pallas as pl
from jax.experimental.pallas import tpu as pltpu
from jax.experimental.pallas import tpu_sc as plsc

BITS = 17
MAXV = 2 ** (BITS - 1) - 1
SIGN = 1 << (BITS - 1)
LANES = 16
NW = 32


def kernel(x, table):
    del table
    B, S = x.shape
    N = B * S
    xf = x.reshape(N).astype(jnp.int32)

    n_w = N // NW
    C = 2048
    CB = C * BITS
    n_it = n_w // C
    n_j = n_it // 2

    mesh = plsc.VectorSubcoreMesh(core_axis_name="c", subcore_axis_name="s")

    @functools.partial(
        pl.kernel,
        mesh=mesh,
        out_type=jax.ShapeDtypeStruct((N * BITS,), jnp.float32),
        scratch_types=[
            pltpu.VMEM((C,), jnp.int32),
            pltpu.VMEM((C,), jnp.int32),
            pltpu.VMEM((CB,), jnp.float32),
            pltpu.VMEM((CB,), jnp.float32),
            pltpu.SemaphoreType.DMA,
            pltpu.SemaphoreType.DMA,
            pltpu.SemaphoreType.DMA,
            pltpu.SemaphoreType.DMA,
        ],
        compiler_params=pltpu.CompilerParams(needs_layout_passes=False),
    )
    def run(x_hbm, out_hbm, xb0, xb1, ob0, ob1, is0, is1, os0, os1):
        wid = lax.axis_index("s") * 2 + lax.axis_index("c")
        base = wid * n_w

        lane = lax.iota(jnp.int32, LANES)
        divs = [(lane + t * LANES) // BITS for t in range(BITS)]
        shifts = [(BITS - 1) - ((lane + t * LANES) % BITS) for t in range(BITS)]

        def in_copy(it, xb, sem):
            return pltpu.make_async_copy(
                x_hbm.at[pl.ds(base + it * C, C)], xb, sem
            )

        def out_copy(it, ob, sem):
            return pltpu.make_async_copy(
                ob, out_hbm.at[pl.ds((base + it * C) * BITS, CB)], sem
            )

        def compute(xb, ob):
            @plsc.parallel_loop(0, C // LANES, unroll=2)
            def body(g):
                sl = pl.ds(g * LANES, LANES)
                xv = xb[sl]
                xc = jnp.clip(xv, -MAXV, MAXV)
                xb[sl] = jnp.abs(xc) + jnp.where(xc < 0, SIGN, 0)
                gb = g * LANES
                obase = g * (LANES * BITS)
                for t in range(BITS):
                    u = plsc.load_gather(xb, [gb + divs[t]])
                    bit = (u >> shifts[t]) & 1
                    ob[pl.ds(obase + t * LANES, LANES)] = bit.astype(
                        jnp.float32
                    )

        in_copy(0, xb0, is0).start()
        in_copy(1, xb1, is1).start()

        def j_body(j, carry):
            for it, xb, ob, isem, osem in (
                (2 * j, xb0, ob0, is0, os0),
                (2 * j + 1, xb1, ob1, is1, os1),
            ):
                in_copy(it, xb, isem).wait()

                @pl.when(j > 0)
                def _wait_out():
                    out_copy(it, ob, osem).wait()

                compute(xb, ob)
                out_copy(it, ob, osem).start()

                @pl.when(j < n_j - 1)
                def _next_in():
                    in_copy(it + 2, xb, isem).start()

            return carry

        lax.fori_loop(0, n_j, j_body, 0)
        out_copy(n_it - 2, ob0, os0).wait()
        out_copy(n_it - 1, ob1, os1).wait()

    out = run(xf)
    return out.reshape(B, S, BITS)


# speedup vs baseline: 481.6450x; 41.0180x over previous
import functools

import jax
import jax.numpy as jnp
from jax import lax
from jax.experimental import pallas as pl
from jax.experimental.pallas import tpu as pltpu
from jax.experimental.pallas import tpu_sc as plsc

BITS = 17
MAXV = 2 ** (BITS - 1) - 1
SIGN = 1 << (BITS - 1)
LANES = 16
NW = 32


def kernel(x, table):
    del table
    B, S = x.shape
    N = B * S
    x_lin = (
        x.astype(jnp.int32)
        .reshape(B // 128, 128, S // 8, 8)
        .transpose(2, 0, 3, 1)
        .reshape(N)
    )

    n_w = N // NW
    C = 3200
    n_it = n_w // C
    n_j = n_it // 2

    mesh = plsc.VectorSubcoreMesh(core_axis_name="c", subcore_axis_name="s")

    @functools.partial(
        pl.kernel,
        mesh=mesh,
        out_type=jax.ShapeDtypeStruct((BITS * N,), jnp.float32),
        scratch_types=[
            pltpu.VMEM((C,), jnp.int32),
            pltpu.VMEM((C,), jnp.int32),
            pltpu.VMEM((BITS * C,), jnp.float32),
            pltpu.VMEM((BITS * C,), jnp.float32),
            pltpu.SemaphoreType.DMA,
            pltpu.SemaphoreType.DMA,
            pltpu.SemaphoreType.DMA,
            pltpu.SemaphoreType.DMA,
        ],
        compiler_params=pltpu.CompilerParams(needs_layout_passes=False),
    )
    def run(x_hbm, out_hbm, xb0, xb1, ob0, ob1, is0, is1, os0, os1):
        wid = lax.axis_index("s") * 2 + lax.axis_index("c")
        base = wid * n_w

        def in_copy(it, xb, sem):
            return pltpu.make_async_copy(
                x_hbm.at[pl.ds(base + it * C, C)], xb, sem
            )

        def out_copies(it, ob, sem):
            return [
                pltpu.make_async_copy(
                    ob.at[pl.ds(k * C, C)],
                    out_hbm.at[pl.ds(k * N + base + it * C, C)],
                    sem,
                )
                for k in range(BITS)
            ]

        def compute(xb, ob):
            @plsc.parallel_loop(0, C // LANES, unroll=4)
            def body(g):
                sl = pl.ds(g * LANES, LANES)
                xv = xb[sl]
                xc = jnp.clip(xv, -MAXV, MAXV)
                u = jnp.abs(xc) + jnp.where(xc < 0, SIGN, 0)
                for k in range(BITS):
                    bit = (u >> (BITS - 1 - k)) & 1
                    ob[pl.ds(k * C + g * LANES, LANES)] = bit.astype(
                        jnp.float32
                    )

        in_copy(0, xb0, is0).start()
        in_copy(1, xb1, is1).start()

        def j_body(j, carry):
            for it, xb, ob, isem, osem in (
                (2 * j, xb0, ob0, is0, os0),
                (2 * j + 1, xb1, ob1, is1, os1),
            ):
                in_copy(it, xb, isem).wait()

                @pl.when(j > 0)
                def _wait_out():
                    for c in out_copies(it, ob, osem):
                        c.wait()

                compute(xb, ob)
                for c in out_copies(it, ob, osem):
                    c.start()

                @pl.when(j < n_j - 1)
                def _next_in():
                    in_copy(it + 2, xb, isem).start()

            return carry

        lax.fori_loop(0, n_j, j_body, 0)
        for c in out_copies(n_it - 2, ob0, os0):
            c.wait()
        for c in out_copies(n_it - 1, ob1, os1):
            c.wait()

    out = run(x_lin)
    y = out.reshape(BITS, S // 8, B // 128, 8, 128)
    return y.transpose(2, 4, 1, 3, 0).reshape(B, S, BITS)


# speedup vs baseline: 485.7412x; 1.0085x over previous
import functools

import jax
import jax.numpy as jnp
from jax import lax
from jax.experimental import pallas as pl
from jax.experimental.pallas import tpu as pltpu
from jax.experimental.pallas import tpu_sc as plsc

BITS = 17
MAXV = 2 ** (BITS - 1) - 1
SIGN = 1 << (BITS - 1)
LANES = 16
NW = 32


def kernel(x, table):
    del table
    B, S = x.shape
    N = B * S
    x_lin = (
        x.astype(jnp.int32)
        .reshape(B // 128, 128, S // 8, 8)
        .transpose(2, 0, 3, 1)
        .reshape(N)
    )

    n_w = N // NW
    C = 3200
    n_it = n_w // C
    n_j = n_it // 2

    mesh = plsc.VectorSubcoreMesh(core_axis_name="c", subcore_axis_name="s")

    @functools.partial(
        pl.kernel,
        mesh=mesh,
        out_type=jax.ShapeDtypeStruct((BITS * N,), jnp.float32),
        scratch_types=[
            pltpu.VMEM((C,), jnp.int32),
            pltpu.VMEM((C,), jnp.int32),
            pltpu.VMEM((BITS * C,), jnp.float32),
            pltpu.VMEM((BITS * C,), jnp.float32),
            pltpu.SemaphoreType.DMA,
            pltpu.SemaphoreType.DMA,
            pltpu.SemaphoreType.DMA,
            pltpu.SemaphoreType.DMA,
        ],
        compiler_params=pltpu.CompilerParams(needs_layout_passes=False),
    )
    def run(x_hbm, out_hbm, xb0, xb1, ob0, ob1, is0, is1, os0, os1):
        wid = lax.axis_index("s") * 2 + lax.axis_index("c")
        base = wid * n_w

        def in_copy(it, xb, sem):
            return pltpu.make_async_copy(
                x_hbm.at[pl.ds(base + it * C, C)], xb, sem
            )

        def out_copies(it, ob, sem):
            return [
                pltpu.make_async_copy(
                    ob.at[pl.ds(k * C, C)],
                    out_hbm.at[pl.ds(k * N + base + it * C, C)],
                    sem,
                )
                for k in range(BITS)
            ]

        def compute(xb, ob):
            @plsc.parallel_loop(0, C // LANES, unroll=8)
            def body(g):
                sl = pl.ds(g * LANES, LANES)
                xv = xb[sl]
                xc = jnp.clip(xv, -MAXV, MAXV)
                u = jnp.abs(xc) + jnp.where(xc < 0, SIGN, 0)
                for k in range(BITS):
                    bit = (u >> (BITS - 1 - k)) & 1
                    ob[pl.ds(k * C + g * LANES, LANES)] = bit.astype(
                        jnp.float32
                    )

        in_copy(0, xb0, is0).start()
        in_copy(1, xb1, is1).start()

        def j_body(j, carry):
            for it, xb, ob, isem, osem in (
                (2 * j, xb0, ob0, is0, os0),
                (2 * j + 1, xb1, ob1, is1, os1),
            ):
                in_copy(it, xb, isem).wait()

                @pl.when(j > 0)
                def _wait_out():
                    for c in out_copies(it, ob, osem):
                        c.wait()

                compute(xb, ob)
                for c in out_copies(it, ob, osem):
                    c.start()

                @pl.when(j < n_j - 1)
                def _next_in():
                    in_copy(it + 2, xb, isem).start()

            return carry

        lax.fori_loop(0, n_j, j_body, 0)
        for c in out_copies(n_it - 2, ob0, os0):
            c.wait()
        for c in out_copies(n_it - 1, ob1, os1):
            c.wait()

    out = run(x_lin)
    y = out.reshape(BITS, S // 8, B // 128, 8, 128)
    return y.transpose(2, 4, 1, 3, 0).reshape(B, S, BITS)
